# Initial kernel scaffold; baseline (speedup 1.0000x reference)
import jax
import jax.numpy as jnp
from jax.experimental import pallas as pl


def kernel(X, edge_idx, edge_weight, edge_attr, batch_map, pW1, pb1, pW2, pb2, pW3, pb3, gW1, gb1, gW2, gb2, gW3, gb3, qW1, qb1, qW2, qb2, qW3, qb3, oW, ob):
    raise NotImplementedError("write your pallas kernel here")


# speedup vs baseline: 6.9600x; 6.9600x over previous
import functools

import jax
import jax.numpy as jnp
from jax import lax
from jax.experimental import pallas as pl
from jax.experimental.pallas import tpu as pltpu
from jax.experimental.pallas import tpu_sc as plsc

_N = 10000
_E = 320000
_H = 64
_B = 64
_NC = 2
_NS = 16
_NT = _NC * _NS
_EPT = _E // _NT
_C = 128
_NFULL = _EPT // _C
_CT = _EPT - _NFULL * _C
_RPT = _N // _NS

_mesh = plsc.VectorSubcoreMesh(
    core_axis_name="c", subcore_axis_name="s", num_cores=_NC, num_subcores=_NS
)


def _mm(a, b, precision=None):
    return lax.dot_general(
        a, b, (((1,), (0,)), ((), ())),
        precision=precision,
        preferred_element_type=jnp.float32,
    )


@functools.partial(
    pl.kernel,
    out_type=jax.ShapeDtypeStruct((_NC * _N,), jnp.float32),
    mesh=_mesh,
    scratch_types=[
        pltpu.VMEM((_C,), jnp.int32),
        pltpu.VMEM((_C,), jnp.float32),
        pltpu.VMEM((_CT,), jnp.int32),
        pltpu.VMEM((_CT,), jnp.float32),
        pltpu.VMEM((640,), jnp.float32),
        pltpu.VMEM_SHARED((_N,), jnp.float32),
    ],
)
def _sc_deg(col, w, out, ci, wv, cit, wvt, zbuf, dagg):
    cid = lax.axis_index("c")
    sid = lax.axis_index("s")
    wid = sid * _NC + cid

    def _z(i, carry):
        zbuf[pl.ds(i * 16, 16)] = jnp.zeros((16,), jnp.float32)
        return carry

    lax.fori_loop(0, 40, _z, None)
    start = sid * 624

    @pl.when(sid == _NS - 1)
    def _():
        pltpu.sync_copy(zbuf, dagg.at[pl.ds(start, 640)])

    @pl.when(sid < _NS - 1)
    def _():
        pltpu.sync_copy(zbuf.at[pl.ds(0, 624)], dagg.at[pl.ds(start, 624)])

    plsc.subcore_barrier()
    ebase = wid * _EPT

    def _chunk(ci_, wv_, base, n):
        pltpu.sync_copy(col.at[pl.ds(base, n)], ci_)
        pltpu.sync_copy(w.at[pl.ds(base, n)], wv_)
        pltpu.sync_copy(wv_, dagg.at[ci_], add=True)

    def _main(i, carry):
        _chunk(ci, wv, ebase + i * _C, _C)
        return carry

    lax.fori_loop(0, _NFULL, _main, None)
    _chunk(cit, wvt, ebase + _NFULL * _C, _CT)
    plsc.subcore_barrier()

    obase = cid * _N + start

    @pl.when(sid == _NS - 1)
    def _():
        pltpu.sync_copy(dagg.at[pl.ds(start, 640)], zbuf)
        pltpu.sync_copy(zbuf, out.at[pl.ds(obase, 640)])

    @pl.when(sid < _NS - 1)
    def _():
        pltpu.sync_copy(dagg.at[pl.ds(start, 624)], zbuf.at[pl.ds(0, 624)])
        pltpu.sync_copy(zbuf.at[pl.ds(0, 624)], out.at[pl.ds(obase, 624)])


@functools.partial(
    pl.kernel,
    out_type=jax.ShapeDtypeStruct((_NC * _N, _H), jnp.float32),
    mesh=_mesh,
    scratch_types=[
        pltpu.VMEM((_C,), jnp.int32),
        pltpu.VMEM((_C,), jnp.int32),
        pltpu.VMEM((_C,), jnp.float32),
        pltpu.VMEM((_C, _H), jnp.float32),
        pltpu.VMEM((_CT,), jnp.int32),
        pltpu.VMEM((_CT,), jnp.int32),
        pltpu.VMEM((_CT,), jnp.float32),
        pltpu.VMEM((_CT, _H), jnp.float32),
        pltpu.VMEM_SHARED((_N, _H), jnp.float32),
        pltpu.SemaphoreType.DMA,
    ],
    compiler_params=pltpu.CompilerParams(use_tc_tiling_on_sc=False),
)
def _sc_scatter(hn, row, col, w, out, ri, ci, wv, rows, rit, cit, wvt, rowst,
                agg, sem):
    cid = lax.axis_index("c")
    sid = lax.axis_index("s")
    wid = sid * _NC + cid

    def _z(i, carry):
        zero = jnp.zeros((16,), jnp.float32)
        for j in range(4):
            rows[i, pl.ds(j * 16, 16)] = zero
        return carry

    lax.fori_loop(0, _C, _z, None)
    nbase = sid * 624

    def _zero_slice(nchunks, rem):
        for k in range(nchunks):
            pltpu.sync_copy(rows, agg.at[pl.ds(nbase + k * _C, _C)])
        if rem:
            pltpu.sync_copy(rows.at[pl.ds(0, rem)],
                            agg.at[pl.ds(nbase + nchunks * _C, rem)])

    @pl.when(sid == _NS - 1)
    def _():
        _zero_slice(5, 0)

    @pl.when(sid < _NS - 1)
    def _():
        _zero_slice(4, 112)

    plsc.subcore_barrier()

    ebase = wid * _EPT

    def _chunk(ri_, ci_, wv_, rows_, base, n):
        pltpu.sync_copy(row.at[pl.ds(base, n)], ri_)
        pltpu.sync_copy(col.at[pl.ds(base, n)], ci_)
        pltpu.sync_copy(w.at[pl.ds(base, n)], wv_)
        pltpu.async_copy(hn.at[ri_], rows_, sem).wait()

        def _grp(g, carry):
            wvec = wv_[pl.ds(g * 16, 16)]
            for i in range(16):
                e = g * 16 + i
                wb = lax.broadcast(wvec[i], (16,))
                for j in range(4):
                    sl = pl.ds(j * 16, 16)
                    rows_[e, sl] = rows_[e, sl] * wb
            return carry

        lax.fori_loop(0, n // 16, _grp, None)
        pltpu.sync_copy(rows_, agg.at[ci_], add=True)

    def _main(i, carry):
        _chunk(ri, ci, wv, rows, ebase + i * _C, _C)
        return carry

    lax.fori_loop(0, _NFULL, _main, None)
    _chunk(rit, cit, wvt, rowst, ebase + _NFULL * _C, _CT)
    plsc.subcore_barrier()
    def _wout(nchunks, rem):
        for k in range(nchunks):
            pltpu.sync_copy(agg.at[pl.ds(nbase + k * _C, _C)], rows)
            pltpu.sync_copy(rows, out.at[pl.ds(cid * _N + nbase + k * _C, _C)])
        if rem:
            pltpu.sync_copy(agg.at[pl.ds(nbase + nchunks * _C, rem)],
                            rows.at[pl.ds(0, rem)])
            pltpu.sync_copy(rows.at[pl.ds(0, rem)],
                            out.at[pl.ds(cid * _N + nbase + nchunks * _C, rem)])

    @pl.when(sid == _NS - 1)
    def _():
        _wout(5, 0)

    @pl.when(sid < _NS - 1)
    def _():
        _wout(4, 112)


def _relu(x):
    return jnp.maximum(x, 0.0)


def _dinv_of(da_ref, db_ref):
    deg = da_ref[...] + db_ref[...]
    safe = jnp.where(deg > 0, deg, 1.0)
    return jnp.where(deg > 0, 1.0 / jnp.sqrt(safe), 0.0)


def _tc_pre_body(x, w1, b1, w2, b2, w3, b3, o):
    h = _relu(_mm(x[...], w1[...]) + b1[...])
    h = _relu(_mm(h, w2[...]) + b2[...])
    o[...] = _relu(_mm(h, w3[...]) + b3[...])


def _tc_scale_body(h0, gw1, da, db, o):
    dinv = _dinv_of(da, db)
    o[...] = _mm(h0[...], gw1[...]) * dinv


def _tc_mid_body(part, da, db, gb, wn, o):
    dinv = _dinv_of(da, db)
    raw = part[0] + part[1]
    h = _relu(raw * dinv + gb[...])
    o[...] = _mm(h, wn[...]) * dinv


def _tc_fin_body(part, da, db, gb3, bm, qw1, qb1, qw2, qb2, qw3, qb3, ow, ob, o):
    dinv = _dinv_of(da, db)
    raw = part[0] + part[1]
    h3 = _relu(raw * dinv + gb3[...])
    seg = lax.broadcasted_iota(jnp.int32, (_B, 1), 0)
    pt = (seg == bm[...]).astype(jnp.float32)
    sums = _mm(pt, h3, precision=lax.Precision.HIGHEST)
    cnt = jnp.sum(pt, axis=1, keepdims=True)
    p = sums / jnp.maximum(cnt, 1.0)
    p = _relu(_mm(p, qw1[...]) + qb1[...])
    p = _relu(_mm(p, qw2[...]) + qb2[...])
    p = _relu(_mm(p, qw3[...]) + qb3[...])
    o[...] = _mm(p, ow[...]) + ob[...]


def _tc(body, out_shape, *args):
    return pl.pallas_call(
        body, out_shape=jax.ShapeDtypeStruct(out_shape, jnp.float32)
    )(*args)


def kernel(X, edge_idx, edge_weight, edge_attr, batch_map,
           pW1, pb1, pW2, pb2, pW3, pb3,
           gW1, gb1, gW2, gb2, gW3, gb3,
           qW1, qb1, qW2, qb2, qW3, qb3, oW, ob):
    row = edge_idx[0]
    col = edge_idx[1]
    w = edge_attr.reshape(-1)

    degp = _sc_deg(col, w).reshape(_NC, _N)
    da = degp[0][:, None]
    db = degp[1][:, None]

    h0 = _tc(_tc_pre_body, (_N, _H), X,
             pW1, pb1.reshape(1, -1), pW2, pb2.reshape(1, -1),
             pW3, pb3.reshape(1, -1))

    hn = _tc(_tc_scale_body, (_N, _H), h0, gW1, da, db)
    p1 = _sc_scatter(hn, row, col, w).reshape(_NC, _N, _H)
    hn = _tc(_tc_mid_body, (_N, _H), p1, da, db, gb1.reshape(1, -1), gW2)
    p2 = _sc_scatter(hn, row, col, w).reshape(_NC, _N, _H)
    hn = _tc(_tc_mid_body, (_N, _H), p2, da, db, gb2.reshape(1, -1), gW3)
    p3 = _sc_scatter(hn, row, col, w).reshape(_NC, _N, _H)

    out = _tc(_tc_fin_body, (_B, 1), p3, da, db, gb3.reshape(1, -1),
              batch_map.reshape(1, -1), qW1, qb1.reshape(1, -1),
              qW2, qb2.reshape(1, -1), qW3, qb3.reshape(1, -1),
              oW, ob.reshape(1, -1))
    return out


# speedup vs baseline: 12.7085x; 1.8259x over previous
import functools

import jax
import jax.numpy as jnp
from jax import lax
from jax.experimental import pallas as pl
from jax.experimental.pallas import tpu as pltpu
from jax.experimental.pallas import tpu_sc as plsc

_N = 10000
_E = 320000
_H = 64
_B = 64
_NC = 2
_NS = 16
_NT = _NC * _NS
_EPT = _E // _NT
_C = 128
_NFULL = _EPT // _C
_CT = _EPT - _NFULL * _C
_RPT = _N // _NS

_mesh = plsc.VectorSubcoreMesh(
    core_axis_name="c", subcore_axis_name="s", num_cores=_NC, num_subcores=_NS
)


def _mm(a, b, precision=None):
    return lax.dot_general(
        a, b, (((1,), (0,)), ((), ())),
        precision=precision,
        preferred_element_type=jnp.float32,
    )


@functools.partial(
    pl.kernel,
    out_type=jax.ShapeDtypeStruct((_NC * _N,), jnp.float32),
    mesh=_mesh,
    scratch_types=[
        pltpu.VMEM((_C,), jnp.int32),
        pltpu.VMEM((_C,), jnp.float32),
        pltpu.VMEM((_CT,), jnp.int32),
        pltpu.VMEM((_CT,), jnp.float32),
        pltpu.VMEM((640,), jnp.float32),
        pltpu.VMEM_SHARED((_N,), jnp.float32),
    ],
)
def _sc_deg(col, w, out, ci, wv, cit, wvt, zbuf, dagg):
    cid = lax.axis_index("c")
    sid = lax.axis_index("s")
    wid = sid * _NC + cid

    def _z(i, carry):
        zbuf[pl.ds(i * 16, 16)] = jnp.zeros((16,), jnp.float32)
        return carry

    lax.fori_loop(0, 40, _z, None)
    start = sid * 624

    @pl.when(sid == _NS - 1)
    def _():
        pltpu.sync_copy(zbuf, dagg.at[pl.ds(start, 640)])

    @pl.when(sid < _NS - 1)
    def _():
        pltpu.sync_copy(zbuf.at[pl.ds(0, 624)], dagg.at[pl.ds(start, 624)])

    plsc.subcore_barrier()
    ebase = wid * _EPT

    def _chunk(ci_, wv_, base, n):
        pltpu.sync_copy(col.at[pl.ds(base, n)], ci_)
        pltpu.sync_copy(w.at[pl.ds(base, n)], wv_)
        pltpu.sync_copy(wv_, dagg.at[ci_], add=True)

    def _main(i, carry):
        _chunk(ci, wv, ebase + i * _C, _C)
        return carry

    lax.fori_loop(0, _NFULL, _main, None)
    _chunk(cit, wvt, ebase + _NFULL * _C, _CT)
    plsc.subcore_barrier()

    obase = cid * _N + start

    @pl.when(sid == _NS - 1)
    def _():
        pltpu.sync_copy(dagg.at[pl.ds(start, 640)], zbuf)
        pltpu.sync_copy(zbuf, out.at[pl.ds(obase, 640)])

    @pl.when(sid < _NS - 1)
    def _():
        pltpu.sync_copy(dagg.at[pl.ds(start, 624)], zbuf.at[pl.ds(0, 624)])
        pltpu.sync_copy(zbuf.at[pl.ds(0, 624)], out.at[pl.ds(obase, 624)])


_C2 = 80
_NCH = _EPT // _C2
_NB = 5
_NG = _NCH // _NB
_ZR = 125


@functools.partial(
    pl.kernel,
    out_type=jax.ShapeDtypeStruct((_NC * _N, _H), jnp.float32),
    mesh=_mesh,
    scratch_types=[
        pltpu.VMEM((_NCH, _C2), jnp.int32),
        pltpu.VMEM((_NCH, _C2), jnp.int32),
        pltpu.VMEM((_NCH, _C2), jnp.float32),
        pltpu.VMEM((_ZR, _H), jnp.float32),
        pltpu.VMEM_SHARED((_N, _H), jnp.float32),
    ]
    + [pltpu.VMEM((_C2, _H), jnp.float32)] * _NB
    + [pltpu.SemaphoreType.DMA] * (2 * _NB),
    compiler_params=pltpu.CompilerParams(use_tc_tiling_on_sc=False),
)
def _sc_scatter(hn, row2, col2, w2, out, rbuf, cbuf, wbuf, zbuf, agg,
                r0, r1, r2, r3, r4, g0, g1, g2, g3, g4, s0, s1, s2, s3, s4):
    cid = lax.axis_index("c")
    sid = lax.axis_index("s")
    wid = sid * _NC + cid
    rows = [r0, r1, r2, r3, r4]
    gsem = [g0, g1, g2, g3, g4]
    ssem = [s0, s1, s2, s3, s4]

    cb = wid * _NCH
    pltpu.sync_copy(row2.at[pl.ds(cb, _NCH)], rbuf)
    pltpu.sync_copy(col2.at[pl.ds(cb, _NCH)], cbuf)
    pltpu.sync_copy(w2.at[pl.ds(cb, _NCH)], wbuf)

    def _z(i, carry):
        zero = jnp.zeros((16,), jnp.float32)
        for j in range(4):
            zbuf[i, pl.ds(j * 16, 16)] = zero
        return carry

    lax.fori_loop(0, _ZR, _z, None)
    nbase = sid * (_N // _NS)
    for k in range(5):
        pltpu.sync_copy(zbuf, agg.at[pl.ds(nbase + k * _ZR, _ZR)])

    pltpu.async_copy(hn.at[rbuf.at[0]], rows[0], gsem[0])
    pltpu.async_copy(hn.at[rbuf.at[1]], rows[1], gsem[1])
    plsc.subcore_barrier()

    def _compute(b, i):
        def _grp(gi, carry):
            wvec = wbuf[i, pl.ds(gi * 16, 16)]
            for l in range(16):
                wb = lax.broadcast(wvec[l], (16,))
                e = gi * 16 + l
                for j in range(4):
                    sl = pl.ds(j * 16, 16)
                    rows[b][e, sl] = rows[b][e, sl] * wb
            return carry

        lax.fori_loop(0, _C2 // 16, _grp, None)

    def _iter(g, b):
        i = g * _NB + b
        nxt = (b + 2) % _NB

        def _drain_nxt():
            pltpu.make_async_copy(rows[nxt], agg.at[cbuf.at[0]],
                                  ssem[nxt]).wait()

        def _prefetch():
            pltpu.async_copy(hn.at[rbuf.at[i + 2]], rows[nxt], gsem[nxt])

        if b < 3:
            @pl.when(g > 0)
            def _():
                _drain_nxt()
            _prefetch()
        else:
            @pl.when(g < _NG - 1)
            def _():
                _drain_nxt()
                _prefetch()

        pltpu.make_async_copy(hn.at[rbuf.at[i]], rows[b], gsem[b]).wait()
        _compute(b, i)
        pltpu.async_copy(rows[b], agg.at[cbuf.at[i]], ssem[b], add=True)

    def _outer(g, carry):
        for b in range(_NB):
            _iter(g, b)
        return carry

    lax.fori_loop(0, _NG, _outer, None)
    for b in range(_NB):
        pltpu.make_async_copy(rows[b], agg.at[cbuf.at[0]], ssem[b]).wait()
    plsc.subcore_barrier()

    for k in range(5):
        pltpu.sync_copy(agg.at[pl.ds(nbase + k * _ZR, _ZR)], zbuf)
        pltpu.sync_copy(zbuf, out.at[pl.ds(cid * _N + nbase + k * _ZR, _ZR)])


def _relu(x):
    return jnp.maximum(x, 0.0)


def _dinv_of(da_ref, db_ref):
    deg = da_ref[...] + db_ref[...]
    safe = jnp.where(deg > 0, deg, 1.0)
    return jnp.where(deg > 0, 1.0 / jnp.sqrt(safe), 0.0)


def _tc_pre_body(x, w1, b1, w2, b2, w3, b3, o):
    h = _relu(_mm(x[...], w1[...]) + b1[...])
    h = _relu(_mm(h, w2[...]) + b2[...])
    o[...] = _relu(_mm(h, w3[...]) + b3[...])


def _tc_scale_body(h0, gw1, da, db, o):
    dinv = _dinv_of(da, db)
    o[...] = _mm(h0[...], gw1[...]) * dinv


def _tc_mid_body(part, da, db, gb, wn, o):
    dinv = _dinv_of(da, db)
    raw = part[0] + part[1]
    h = _relu(raw * dinv + gb[...])
    o[...] = _mm(h, wn[...]) * dinv


def _tc_fin_body(part, da, db, gb3, bm, qw1, qb1, qw2, qb2, qw3, qb3, ow, ob, o):
    dinv = _dinv_of(da, db)
    raw = part[0] + part[1]
    h3 = _relu(raw * dinv + gb3[...])
    seg = lax.broadcasted_iota(jnp.int32, (_B, 1), 0)
    pt = (seg == bm[...]).astype(jnp.float32)
    sums = _mm(pt, h3, precision=lax.Precision.HIGHEST)
    cnt = jnp.sum(pt, axis=1, keepdims=True)
    p = sums / jnp.maximum(cnt, 1.0)
    p = _relu(_mm(p, qw1[...]) + qb1[...])
    p = _relu(_mm(p, qw2[...]) + qb2[...])
    p = _relu(_mm(p, qw3[...]) + qb3[...])
    o[...] = _mm(p, ow[...]) + ob[...]


def _tc(body, out_shape, *args):
    return pl.pallas_call(
        body, out_shape=jax.ShapeDtypeStruct(out_shape, jnp.float32)
    )(*args)


def kernel(X, edge_idx, edge_weight, edge_attr, batch_map,
           pW1, pb1, pW2, pb2, pW3, pb3,
           gW1, gb1, gW2, gb2, gW3, gb3,
           qW1, qb1, qW2, qb2, qW3, qb3, oW, ob):
    row = edge_idx[0]
    col = edge_idx[1]
    w = edge_attr.reshape(-1)
    row2 = row.reshape(_E // _C2, _C2)
    col2 = col.reshape(_E // _C2, _C2)
    w2 = w.reshape(_E // _C2, _C2)

    degp = _sc_deg(col, w).reshape(_NC, _N)
    da = degp[0][:, None]
    db = degp[1][:, None]

    h0 = _tc(_tc_pre_body, (_N, _H), X,
             pW1, pb1.reshape(1, -1), pW2, pb2.reshape(1, -1),
             pW3, pb3.reshape(1, -1))

    hn = _tc(_tc_scale_body, (_N, _H), h0, gW1, da, db)
    p1 = _sc_scatter(hn, row2, col2, w2).reshape(_NC, _N, _H)
    hn = _tc(_tc_mid_body, (_N, _H), p1, da, db, gb1.reshape(1, -1), gW2)
    p2 = _sc_scatter(hn, row2, col2, w2).reshape(_NC, _N, _H)
    hn = _tc(_tc_mid_body, (_N, _H), p2, da, db, gb2.reshape(1, -1), gW3)
    p3 = _sc_scatter(hn, row2, col2, w2).reshape(_NC, _N, _H)

    out = _tc(_tc_fin_body, (_B, 1), p3, da, db, gb3.reshape(1, -1),
              batch_map.reshape(1, -1), qW1, qb1.reshape(1, -1),
              qW2, qb2.reshape(1, -1), qW3, qb3.reshape(1, -1),
              oW, ob.reshape(1, -1))
    return out


# speedup vs baseline: 23.8162x; 1.8740x over previous
import functools

import jax
import jax.numpy as jnp
from jax import lax
from jax.experimental import pallas as pl
from jax.experimental.pallas import tpu as pltpu
from jax.experimental.pallas import tpu_sc as plsc

_N = 10000
_E = 320000
_H = 64
_B = 64
_NC = 2
_NS = 16
_NT = _NC * _NS
_EPT = _E // _NT
_C = 128
_NFULL = _EPT // _C
_CT = _EPT - _NFULL * _C
_RPT = _N // _NS

_mesh = plsc.VectorSubcoreMesh(
    core_axis_name="c", subcore_axis_name="s", num_cores=_NC, num_subcores=_NS
)


def _mm(a, b, precision=None):
    return lax.dot_general(
        a, b, (((1,), (0,)), ((), ())),
        precision=precision,
        preferred_element_type=jnp.float32,
    )


@functools.partial(
    pl.kernel,
    out_type=jax.ShapeDtypeStruct((_NC * _N,), jnp.float32),
    mesh=_mesh,
    scratch_types=[
        pltpu.VMEM((125, 80), jnp.int32),
        pltpu.VMEM((125, 80), jnp.float32),
        pltpu.VMEM((640,), jnp.float32),
        pltpu.VMEM_SHARED((_N,), jnp.float32),
    ]
    + [pltpu.SemaphoreType.DMA] * 4,
    compiler_params=pltpu.CompilerParams(use_tc_tiling_on_sc=False),
)
def _sc_deg(col2, w2, out, cbuf, wbuf, zbuf, dagg, s0, s1, s2, s3):
    cid = lax.axis_index("c")
    sid = lax.axis_index("s")
    wid = sid * _NC + cid
    ssem = [s0, s1, s2, s3]

    cb = wid * 125
    pltpu.sync_copy(col2.at[pl.ds(cb, 125)], cbuf)
    pltpu.sync_copy(w2.at[pl.ds(cb, 125)], wbuf)

    def _z(i, carry):
        zbuf[pl.ds(i * 16, 16)] = jnp.zeros((16,), jnp.float32)
        return carry

    lax.fori_loop(0, 40, _z, None)
    start = sid * 624

    @pl.when(sid == _NS - 1)
    def _():
        pltpu.sync_copy(zbuf, dagg.at[pl.ds(start, 640)])

    @pl.when(sid < _NS - 1)
    def _():
        pltpu.sync_copy(zbuf.at[pl.ds(0, 624)], dagg.at[pl.ds(start, 624)])

    plsc.subcore_barrier()

    def _outer(g, carry):
        for b in range(4):
            i = g * 4 + b

            @pl.when(g > 0)
            def _():
                pltpu.make_async_copy(wbuf.at[0], dagg.at[cbuf.at[0]],
                                      ssem[b]).wait()

            pltpu.async_copy(wbuf.at[i], dagg.at[cbuf.at[i]], ssem[b],
                             add=True)
        return carry

    lax.fori_loop(0, 31, _outer, None)
    pltpu.make_async_copy(wbuf.at[0], dagg.at[cbuf.at[0]], ssem[0]).wait()
    pltpu.async_copy(wbuf.at[124], dagg.at[cbuf.at[124]], ssem[0], add=True)
    for b in range(4):
        pltpu.make_async_copy(wbuf.at[0], dagg.at[cbuf.at[0]], ssem[b]).wait()
    plsc.subcore_barrier()

    obase = cid * _N + start

    @pl.when(sid == _NS - 1)
    def _():
        pltpu.sync_copy(dagg.at[pl.ds(start, 640)], zbuf)
        pltpu.sync_copy(zbuf, out.at[pl.ds(obase, 640)])

    @pl.when(sid < _NS - 1)
    def _():
        pltpu.sync_copy(dagg.at[pl.ds(start, 624)], zbuf.at[pl.ds(0, 624)])
        pltpu.sync_copy(zbuf.at[pl.ds(0, 624)], out.at[pl.ds(obase, 624)])


_C2 = 80
_NCH = _EPT // _C2
_NB = 5
_NG = _NCH // _NB
_ZR = 125


@functools.partial(
    pl.kernel,
    out_type=jax.ShapeDtypeStruct((_NC * _N, _H), jnp.float32),
    mesh=_mesh,
    scratch_types=[
        pltpu.VMEM((_NCH, _C2), jnp.int32),
        pltpu.VMEM((_NCH, _C2), jnp.int32),
        pltpu.VMEM((_NCH, _C2), jnp.float32),
        pltpu.VMEM((_ZR, _H), jnp.float32),
        pltpu.VMEM_SHARED((_N, _H), jnp.float32),
    ]
    + [pltpu.VMEM((_C2, _H), jnp.float32)] * _NB
    + [pltpu.SemaphoreType.DMA] * (2 * _NB),
    compiler_params=pltpu.CompilerParams(use_tc_tiling_on_sc=False),
)
def _sc_scatter(hn, row2, col2, w2, out, rbuf, cbuf, wbuf, zbuf, agg,
                r0, r1, r2, r3, r4, g0, g1, g2, g3, g4, s0, s1, s2, s3, s4):
    cid = lax.axis_index("c")
    sid = lax.axis_index("s")
    wid = sid * _NC + cid
    rows = [r0, r1, r2, r3, r4]
    gsem = [g0, g1, g2, g3, g4]
    ssem = [s0, s1, s2, s3, s4]

    cb = wid * _NCH
    pltpu.sync_copy(row2.at[pl.ds(cb, _NCH)], rbuf)
    pltpu.sync_copy(col2.at[pl.ds(cb, _NCH)], cbuf)
    pltpu.sync_copy(w2.at[pl.ds(cb, _NCH)], wbuf)

    def _z(i, carry):
        zero = jnp.zeros((16,), jnp.float32)
        for j in range(4):
            zbuf[i, pl.ds(j * 16, 16)] = zero
        return carry

    lax.fori_loop(0, _ZR, _z, None)
    nbase = sid * (_N // _NS)
    for k in range(5):
        pltpu.sync_copy(zbuf, agg.at[pl.ds(nbase + k * _ZR, _ZR)])

    pltpu.async_copy(hn.at[rbuf.at[0]], rows[0], gsem[0])
    pltpu.async_copy(hn.at[rbuf.at[1]], rows[1], gsem[1])
    plsc.subcore_barrier()

    def _compute(b, i):
        for gi in range(_C2 // 16):
            wvec = wbuf[i, pl.ds(gi * 16, 16)]
            for l in range(16):
                wb = lax.broadcast(wvec[l], (16,))
                e = gi * 16 + l
                for j in range(4):
                    sl = pl.ds(j * 16, 16)
                    rows[b][e, sl] = rows[b][e, sl] * wb

    def _iter(g, b):
        i = g * _NB + b
        nxt = (b + 2) % _NB

        def _drain_nxt():
            pltpu.make_async_copy(rows[nxt], agg.at[cbuf.at[0]],
                                  ssem[nxt]).wait()

        def _prefetch():
            pltpu.async_copy(hn.at[rbuf.at[i + 2]], rows[nxt], gsem[nxt])

        if b < 3:
            @pl.when(g > 0)
            def _():
                _drain_nxt()
            _prefetch()
        else:
            @pl.when(g < _NG - 1)
            def _():
                _drain_nxt()
                _prefetch()

        pltpu.make_async_copy(hn.at[rbuf.at[i]], rows[b], gsem[b]).wait()
        _compute(b, i)
        pltpu.async_copy(rows[b], agg.at[cbuf.at[i]], ssem[b], add=True)

    def _outer(g, carry):
        for b in range(_NB):
            _iter(g, b)
        return carry

    lax.fori_loop(0, _NG, _outer, None)
    for b in range(_NB):
        pltpu.make_async_copy(rows[b], agg.at[cbuf.at[0]], ssem[b]).wait()
    plsc.subcore_barrier()

    for k in range(5):
        pltpu.sync_copy(agg.at[pl.ds(nbase + k * _ZR, _ZR)], zbuf)
        pltpu.sync_copy(zbuf, out.at[pl.ds(cid * _N + nbase + k * _ZR, _ZR)])


def _relu(x):
    return jnp.maximum(x, 0.0)


def _dinv_of(da_ref, db_ref):
    deg = da_ref[...] + db_ref[...]
    safe = jnp.where(deg > 0, deg, 1.0)
    return jnp.where(deg > 0, 1.0 / jnp.sqrt(safe), 0.0)


def _tc_pre_body(x, w1, b1, w2, b2, w3, b3, o):
    h = _relu(_mm(x[...], w1[...]) + b1[...])
    h = _relu(_mm(h, w2[...]) + b2[...])
    o[...] = _relu(_mm(h, w3[...]) + b3[...])


def _tc_scale_body(h0, gw1, da, db, o):
    dinv = _dinv_of(da, db)
    o[...] = _mm(h0[...], gw1[...]) * dinv


def _tc_mid_body(part, da, db, gb, wn, o):
    dinv = _dinv_of(da, db)
    raw = part[0] + part[1]
    h = _relu(raw * dinv + gb[...])
    o[...] = _mm(h, wn[...]) * dinv


def _tc_fin_body(part, da, db, gb3, bm, qw1, qb1, qw2, qb2, qw3, qb3, ow, ob, o):
    dinv = _dinv_of(da, db)
    raw = part[0] + part[1]
    h3 = _relu(raw * dinv + gb3[...])
    seg = lax.broadcasted_iota(jnp.int32, (_B, 1), 0)
    pt = (seg == bm[...]).astype(jnp.float32)
    sums = _mm(pt, h3, precision=lax.Precision.HIGHEST)
    cnt = jnp.sum(pt, axis=1, keepdims=True)
    p = sums / jnp.maximum(cnt, 1.0)
    p = _relu(_mm(p, qw1[...]) + qb1[...])
    p = _relu(_mm(p, qw2[...]) + qb2[...])
    p = _relu(_mm(p, qw3[...]) + qb3[...])
    o[...] = _mm(p, ow[...]) + ob[...]


def _tc(body, out_shape, *args):
    return pl.pallas_call(
        body, out_shape=jax.ShapeDtypeStruct(out_shape, jnp.float32)
    )(*args)


def kernel(X, edge_idx, edge_weight, edge_attr, batch_map,
           pW1, pb1, pW2, pb2, pW3, pb3,
           gW1, gb1, gW2, gb2, gW3, gb3,
           qW1, qb1, qW2, qb2, qW3, qb3, oW, ob):
    row = edge_idx[0]
    col = edge_idx[1]
    w = edge_attr.reshape(-1)
    row2 = row.reshape(_E // _C2, _C2)
    col2 = col.reshape(_E // _C2, _C2)
    w2 = w.reshape(_E // _C2, _C2)

    degp = _sc_deg(col2, w2).reshape(_NC, _N)
    da = degp[0][:, None]
    db = degp[1][:, None]

    h0 = _tc(_tc_pre_body, (_N, _H), X,
             pW1, pb1.reshape(1, -1), pW2, pb2.reshape(1, -1),
             pW3, pb3.reshape(1, -1))

    hn = _tc(_tc_scale_body, (_N, _H), h0, gW1, da, db)
    p1 = _sc_scatter(hn, row2, col2, w2).reshape(_NC, _N, _H)
    hn = _tc(_tc_mid_body, (_N, _H), p1, da, db, gb1.reshape(1, -1), gW2)
    p2 = _sc_scatter(hn, row2, col2, w2).reshape(_NC, _N, _H)
    hn = _tc(_tc_mid_body, (_N, _H), p2, da, db, gb2.reshape(1, -1), gW3)
    p3 = _sc_scatter(hn, row2, col2, w2).reshape(_NC, _N, _H)

    out = _tc(_tc_fin_body, (_B, 1), p3, da, db, gb3.reshape(1, -1),
              batch_map.reshape(1, -1), qW1, qb1.reshape(1, -1),
              qW2, qb2.reshape(1, -1), qW3, qb3.reshape(1, -1),
              oW, ob.reshape(1, -1))
    return out


# speedup vs baseline: 23.9239x; 1.0045x over previous
import functools

import jax
import jax.numpy as jnp
from jax import lax
from jax.experimental import pallas as pl
from jax.experimental.pallas import tpu as pltpu
from jax.experimental.pallas import tpu_sc as plsc

_N = 10000
_E = 320000
_H = 64
_B = 64
_NC = 2
_NS = 16
_NT = _NC * _NS
_EPT = _E // _NT
_C = 128
_NFULL = _EPT // _C
_CT = _EPT - _NFULL * _C
_RPT = _N // _NS

_mesh = plsc.VectorSubcoreMesh(
    core_axis_name="c", subcore_axis_name="s", num_cores=_NC, num_subcores=_NS
)


def _mm(a, b, precision=None):
    return lax.dot_general(
        a, b, (((1,), (0,)), ((), ())),
        precision=precision,
        preferred_element_type=jnp.float32,
    )


@functools.partial(
    pl.kernel,
    out_type=jax.ShapeDtypeStruct((_NC * _N,), jnp.float32),
    mesh=_mesh,
    scratch_types=[
        pltpu.VMEM((125, 80), jnp.int32),
        pltpu.VMEM((125, 80), jnp.float32),
        pltpu.VMEM((640,), jnp.float32),
        pltpu.VMEM_SHARED((_N,), jnp.float32),
    ]
    + [pltpu.SemaphoreType.DMA] * 4,
    compiler_params=pltpu.CompilerParams(use_tc_tiling_on_sc=False),
)
def _sc_deg(col2, w2, out, cbuf, wbuf, zbuf, dagg, s0, s1, s2, s3):
    cid = lax.axis_index("c")
    sid = lax.axis_index("s")
    wid = sid * _NC + cid
    ssem = [s0, s1, s2, s3]

    cb = wid * 125
    pltpu.sync_copy(col2.at[pl.ds(cb, 125)], cbuf)
    pltpu.sync_copy(w2.at[pl.ds(cb, 125)], wbuf)

    def _z(i, carry):
        zbuf[pl.ds(i * 16, 16)] = jnp.zeros((16,), jnp.float32)
        return carry

    lax.fori_loop(0, 40, _z, None)
    start = sid * 624

    @pl.when(sid == _NS - 1)
    def _():
        pltpu.sync_copy(zbuf, dagg.at[pl.ds(start, 640)])

    @pl.when(sid < _NS - 1)
    def _():
        pltpu.sync_copy(zbuf.at[pl.ds(0, 624)], dagg.at[pl.ds(start, 624)])

    plsc.subcore_barrier()

    def _outer(g, carry):
        for b in range(4):
            i = g * 4 + b

            @pl.when(g > 0)
            def _():
                pltpu.make_async_copy(wbuf.at[0], dagg.at[cbuf.at[0]],
                                      ssem[b]).wait()

            pltpu.async_copy(wbuf.at[i], dagg.at[cbuf.at[i]], ssem[b],
                             add=True)
        return carry

    lax.fori_loop(0, 31, _outer, None)
    pltpu.make_async_copy(wbuf.at[0], dagg.at[cbuf.at[0]], ssem[0]).wait()
    pltpu.async_copy(wbuf.at[124], dagg.at[cbuf.at[124]], ssem[0], add=True)
    for b in range(4):
        pltpu.make_async_copy(wbuf.at[0], dagg.at[cbuf.at[0]], ssem[b]).wait()
    plsc.subcore_barrier()

    obase = cid * _N + start

    @pl.when(sid == _NS - 1)
    def _():
        pltpu.sync_copy(dagg.at[pl.ds(start, 640)], zbuf)
        pltpu.sync_copy(zbuf, out.at[pl.ds(obase, 640)])

    @pl.when(sid < _NS - 1)
    def _():
        pltpu.sync_copy(dagg.at[pl.ds(start, 624)], zbuf.at[pl.ds(0, 624)])
        pltpu.sync_copy(zbuf.at[pl.ds(0, 624)], out.at[pl.ds(obase, 624)])


_C2 = 80
_NCH = _EPT // _C2
_NB = 5
_NG = _NCH // _NB
_ZR = 125


@functools.partial(
    pl.kernel,
    out_type=jax.ShapeDtypeStruct((_NC * _N, _H), jnp.float32),
    mesh=_mesh,
    scratch_types=[
        pltpu.VMEM((_NCH, _C2), jnp.int32),
        pltpu.VMEM((_NCH, _C2), jnp.int32),
        pltpu.VMEM((_NCH, _C2), jnp.float32),
        pltpu.VMEM((_ZR, _H), jnp.float32),
        pltpu.VMEM_SHARED((_N, _H), jnp.float32),
    ]
    + [pltpu.VMEM((_C2, _H), jnp.float32)] * _NB
    + [pltpu.SemaphoreType.DMA] * (2 * _NB),
    compiler_params=pltpu.CompilerParams(use_tc_tiling_on_sc=False),
)
def _sc_scatter(hn, row2, col2, w2, out, rbuf, cbuf, wbuf, zbuf, agg,
                r0, r1, r2, r3, r4, g0, g1, g2, g3, g4, s0, s1, s2, s3, s4):
    cid = lax.axis_index("c")
    sid = lax.axis_index("s")
    wid = sid * _NC + cid
    rows = [r0, r1, r2, r3, r4]
    gsem = [g0, g1, g2, g3, g4]
    ssem = [s0, s1, s2, s3, s4]

    cb = wid * _NCH
    pltpu.sync_copy(row2.at[pl.ds(cb, _NCH)], rbuf)
    pltpu.sync_copy(col2.at[pl.ds(cb, _NCH)], cbuf)
    pltpu.sync_copy(w2.at[pl.ds(cb, _NCH)], wbuf)

    def _z(i, carry):
        zero = jnp.zeros((16,), jnp.float32)
        for j in range(4):
            zbuf[i, pl.ds(j * 16, 16)] = zero
        return carry

    lax.fori_loop(0, _ZR, _z, None)
    nbase = sid * (_N // _NS)
    for k in range(5):
        pltpu.sync_copy(zbuf, agg.at[pl.ds(nbase + k * _ZR, _ZR)])

    pltpu.async_copy(hn.at[rbuf.at[0]], rows[0], gsem[0])
    pltpu.async_copy(hn.at[rbuf.at[1]], rows[1], gsem[1])
    pltpu.async_copy(hn.at[rbuf.at[2]], rows[2], gsem[2])
    plsc.subcore_barrier()

    def _compute(b, i):
        for gi in range(_C2 // 16):
            wvec = wbuf[i, pl.ds(gi * 16, 16)]
            for l in range(16):
                wb = lax.broadcast(wvec[l], (16,))
                e = gi * 16 + l
                for j in range(4):
                    sl = pl.ds(j * 16, 16)
                    rows[b][e, sl] = rows[b][e, sl] * wb

    def _iter(g, b):
        i = g * _NB + b
        nxt = (b + 3) % _NB

        def _drain_nxt():
            pltpu.make_async_copy(rows[nxt], agg.at[cbuf.at[0]],
                                  ssem[nxt]).wait()

        def _prefetch():
            pltpu.async_copy(hn.at[rbuf.at[i + 3]], rows[nxt], gsem[nxt])

        if b < 2:
            @pl.when(g > 0)
            def _():
                _drain_nxt()
            _prefetch()
        else:
            @pl.when(g < _NG - 1)
            def _():
                _drain_nxt()
                _prefetch()

        pltpu.make_async_copy(hn.at[rbuf.at[i]], rows[b], gsem[b]).wait()
        _compute(b, i)
        pltpu.async_copy(rows[b], agg.at[cbuf.at[i]], ssem[b], add=True)

    def _outer(g, carry):
        for b in range(_NB):
            _iter(g, b)
        return carry

    lax.fori_loop(0, _NG, _outer, None)
    for b in range(_NB):
        pltpu.make_async_copy(rows[b], agg.at[cbuf.at[0]], ssem[b]).wait()
    plsc.subcore_barrier()

    for k in range(5):
        pltpu.sync_copy(agg.at[pl.ds(nbase + k * _ZR, _ZR)], zbuf)
        pltpu.sync_copy(zbuf, out.at[pl.ds(cid * _N + nbase + k * _ZR, _ZR)])


def _relu(x):
    return jnp.maximum(x, 0.0)


def _dinv_of(da_ref, db_ref):
    deg = da_ref[...] + db_ref[...]
    safe = jnp.where(deg > 0, deg, 1.0)
    return jnp.where(deg > 0, 1.0 / jnp.sqrt(safe), 0.0)


def _tc_pre_body(x, w1, b1, w2, b2, w3, b3, gw1, da, db, o):
    h = _relu(_mm(x[...], w1[...]) + b1[...])
    h = _relu(_mm(h, w2[...]) + b2[...])
    h = _relu(_mm(h, w3[...]) + b3[...])
    dinv = _dinv_of(da, db)
    o[...] = _mm(h, gw1[...]) * dinv


def _tc_mid_body(part, da, db, gb, wn, o):
    dinv = _dinv_of(da, db)
    raw = part[0] + part[1]
    h = _relu(raw * dinv + gb[...])
    o[...] = _mm(h, wn[...]) * dinv


def _tc_fin_body(part, da, db, gb3, bm, qw1, qb1, qw2, qb2, qw3, qb3, ow, ob, o):
    dinv = _dinv_of(da, db)
    raw = part[0] + part[1]
    h3 = _relu(raw * dinv + gb3[...])
    seg = lax.broadcasted_iota(jnp.int32, (_B, 1), 0)
    pt = (seg == bm[...]).astype(jnp.float32)
    sums = _mm(pt, h3, precision=lax.Precision.HIGHEST)
    cnt = jnp.sum(pt, axis=1, keepdims=True)
    p = sums / jnp.maximum(cnt, 1.0)
    p = _relu(_mm(p, qw1[...]) + qb1[...])
    p = _relu(_mm(p, qw2[...]) + qb2[...])
    p = _relu(_mm(p, qw3[...]) + qb3[...])
    o[...] = _mm(p, ow[...]) + ob[...]


def _tc(body, out_shape, *args):
    return pl.pallas_call(
        body, out_shape=jax.ShapeDtypeStruct(out_shape, jnp.float32)
    )(*args)


def kernel(X, edge_idx, edge_weight, edge_attr, batch_map,
           pW1, pb1, pW2, pb2, pW3, pb3,
           gW1, gb1, gW2, gb2, gW3, gb3,
           qW1, qb1, qW2, qb2, qW3, qb3, oW, ob):
    row = edge_idx[0]
    col = edge_idx[1]
    w = edge_attr.reshape(-1)
    row2 = row.reshape(_E // _C2, _C2)
    col2 = col.reshape(_E // _C2, _C2)
    w2 = w.reshape(_E // _C2, _C2)

    degp = _sc_deg(col2, w2).reshape(_NC, _N)
    da = degp[0][:, None]
    db = degp[1][:, None]

    hn = _tc(_tc_pre_body, (_N, _H), X,
             pW1, pb1.reshape(1, -1), pW2, pb2.reshape(1, -1),
             pW3, pb3.reshape(1, -1), gW1, da, db)
    p1 = _sc_scatter(hn, row2, col2, w2).reshape(_NC, _N, _H)
    hn = _tc(_tc_mid_body, (_N, _H), p1, da, db, gb1.reshape(1, -1), gW2)
    p2 = _sc_scatter(hn, row2, col2, w2).reshape(_NC, _N, _H)
    hn = _tc(_tc_mid_body, (_N, _H), p2, da, db, gb2.reshape(1, -1), gW3)
    p3 = _sc_scatter(hn, row2, col2, w2).reshape(_NC, _N, _H)

    out = _tc(_tc_fin_body, (_B, 1), p3, da, db, gb3.reshape(1, -1),
              batch_map.reshape(1, -1), qW1, qb1.reshape(1, -1),
              qW2, qb2.reshape(1, -1), qW3, qb3.reshape(1, -1),
              oW, ob.reshape(1, -1))
    return out


# speedup vs baseline: 24.2108x; 1.0120x over previous
import functools

import jax
import jax.numpy as jnp
from jax import lax
from jax.experimental import pallas as pl
from jax.experimental.pallas import tpu as pltpu
from jax.experimental.pallas import tpu_sc as plsc

_N = 10000
_E = 320000
_H = 64
_B = 64
_NC = 2
_NS = 16
_NT = _NC * _NS
_EPT = _E // _NT
_C = 128
_NFULL = _EPT // _C
_CT = _EPT - _NFULL * _C
_RPT = _N // _NS

_mesh = plsc.VectorSubcoreMesh(
    core_axis_name="c", subcore_axis_name="s", num_cores=_NC, num_subcores=_NS
)


def _mm(a, b, precision=None):
    return lax.dot_general(
        a, b, (((1,), (0,)), ((), ())),
        precision=precision,
        preferred_element_type=jnp.float32,
    )


@functools.partial(
    pl.kernel,
    out_type=jax.ShapeDtypeStruct((_NC * _N,), jnp.float32),
    mesh=_mesh,
    scratch_types=[
        pltpu.VMEM((125, 80), jnp.int32),
        pltpu.VMEM((125, 80), jnp.float32),
        pltpu.VMEM((640,), jnp.float32),
        pltpu.VMEM_SHARED((_N,), jnp.float32),
    ]
    + [pltpu.SemaphoreType.DMA] * 4,
    compiler_params=pltpu.CompilerParams(use_tc_tiling_on_sc=False),
)
def _sc_deg(col2, w2, out, cbuf, wbuf, zbuf, dagg, s0, s1, s2, s3):
    cid = lax.axis_index("c")
    sid = lax.axis_index("s")
    wid = sid * _NC + cid
    ssem = [s0, s1, s2, s3]

    cb = wid * 125
    pltpu.sync_copy(col2.at[pl.ds(cb, 125)], cbuf)
    pltpu.sync_copy(w2.at[pl.ds(cb, 125)], wbuf)

    def _z(i, carry):
        zbuf[pl.ds(i * 16, 16)] = jnp.zeros((16,), jnp.float32)
        return carry

    lax.fori_loop(0, 40, _z, None)
    start = sid * 624

    @pl.when(sid == _NS - 1)
    def _():
        pltpu.sync_copy(zbuf, dagg.at[pl.ds(start, 640)])

    @pl.when(sid < _NS - 1)
    def _():
        pltpu.sync_copy(zbuf.at[pl.ds(0, 624)], dagg.at[pl.ds(start, 624)])

    plsc.subcore_barrier()

    def _outer(g, carry):
        for b in range(4):
            i = g * 4 + b

            @pl.when(g > 0)
            def _():
                pltpu.make_async_copy(wbuf.at[0], dagg.at[cbuf.at[0]],
                                      ssem[b]).wait()

            pltpu.async_copy(wbuf.at[i], dagg.at[cbuf.at[i]], ssem[b],
                             add=True)
        return carry

    lax.fori_loop(0, 31, _outer, None)
    pltpu.make_async_copy(wbuf.at[0], dagg.at[cbuf.at[0]], ssem[0]).wait()
    pltpu.async_copy(wbuf.at[124], dagg.at[cbuf.at[124]], ssem[0], add=True)
    for b in range(4):
        pltpu.make_async_copy(wbuf.at[0], dagg.at[cbuf.at[0]], ssem[b]).wait()
    plsc.subcore_barrier()

    obase = cid * _N + start

    @pl.when(sid == _NS - 1)
    def _():
        pltpu.sync_copy(dagg.at[pl.ds(start, 640)], zbuf)
        pltpu.sync_copy(zbuf, out.at[pl.ds(obase, 640)])

    @pl.when(sid < _NS - 1)
    def _():
        pltpu.sync_copy(dagg.at[pl.ds(start, 624)], zbuf.at[pl.ds(0, 624)])
        pltpu.sync_copy(zbuf.at[pl.ds(0, 624)], out.at[pl.ds(obase, 624)])


_C2 = 80
_NCH = _EPT // _C2
_NB = 5
_NG = _NCH // _NB
_ZR = 125


@functools.partial(
    pl.kernel,
    out_type=jax.ShapeDtypeStruct((_NC * _N, _H), jnp.float32),
    mesh=_mesh,
    scratch_types=[
        pltpu.VMEM((_NCH, _C2), jnp.int32),
        pltpu.VMEM((_NCH, _C2), jnp.int32),
        pltpu.VMEM((_NCH, _C2), jnp.float32),
        pltpu.VMEM((_ZR, _H), jnp.float32),
        pltpu.VMEM_SHARED((_N, _H), jnp.float32),
    ]
    + [pltpu.VMEM((_C2, _H), jnp.float32)] * _NB
    + [pltpu.SemaphoreType.DMA] * (2 * _NB),
    compiler_params=pltpu.CompilerParams(use_tc_tiling_on_sc=False),
)
def _sc_scatter(hn, row2, col2, w2, out, rbuf, cbuf, wbuf, zbuf, agg,
                r0, r1, r2, r3, r4, g0, g1, g2, g3, g4, s0, s1, s2, s3, s4):
    cid = lax.axis_index("c")
    sid = lax.axis_index("s")
    wid = sid * _NC + cid
    rows = [r0, r1, r2, r3, r4]
    gsem = [g0, g1, g2, g3, g4]
    ssem = [s0, s1, s2, s3, s4]

    cb = wid * _NCH
    pltpu.sync_copy(row2.at[pl.ds(cb, _NCH)], rbuf)
    pltpu.sync_copy(col2.at[pl.ds(cb, _NCH)], cbuf)
    pltpu.sync_copy(w2.at[pl.ds(cb, _NCH)], wbuf)

    def _z(i, carry):
        zero = jnp.zeros((16,), jnp.float32)
        for j in range(4):
            zbuf[i, pl.ds(j * 16, 16)] = zero
        return carry

    lax.fori_loop(0, _ZR, _z, None)
    nbase = sid * (_N // _NS)
    for k in range(5):
        pltpu.sync_copy(zbuf, agg.at[pl.ds(nbase + k * _ZR, _ZR)])

    pltpu.async_copy(hn.at[rbuf.at[0]], rows[0], gsem[0])
    pltpu.async_copy(hn.at[rbuf.at[1]], rows[1], gsem[1])
    pltpu.async_copy(hn.at[rbuf.at[2]], rows[2], gsem[2])
    plsc.subcore_barrier()

    def _compute(b, i):
        for gi in range(_C2 // 16):
            wvec = wbuf[i, pl.ds(gi * 16, 16)]
            for l in range(16):
                wb = lax.broadcast(wvec[l], (16,))
                e = gi * 16 + l
                for j in range(4):
                    sl = pl.ds(j * 16, 16)
                    rows[b][e, sl] = rows[b][e, sl] * wb

    def _iter(g, b):
        i = g * _NB + b
        nxt = (b + 3) % _NB

        def _drain_nxt():
            pltpu.make_async_copy(rows[nxt], agg.at[cbuf.at[0]],
                                  ssem[nxt]).wait()

        def _prefetch():
            pltpu.async_copy(hn.at[rbuf.at[i + 3]], rows[nxt], gsem[nxt])

        if b < 2:
            @pl.when(g > 0)
            def _():
                _drain_nxt()
            _prefetch()
        else:
            @pl.when(g < _NG - 1)
            def _():
                _drain_nxt()
                _prefetch()

        pltpu.make_async_copy(hn.at[rbuf.at[i]], rows[b], gsem[b]).wait()
        _compute(b, i)
        pltpu.async_copy(rows[b], agg.at[cbuf.at[i]], ssem[b], add=True)

    def _outer(g, carry):
        for b in range(_NB):
            _iter(g, b)
        return carry

    lax.fori_loop(0, _NG, _outer, None)
    for b in range(_NB):
        pltpu.make_async_copy(rows[b], agg.at[cbuf.at[0]], ssem[b]).wait()
    plsc.subcore_barrier()

    for k in range(5):
        pltpu.sync_copy(agg.at[pl.ds(nbase + k * _ZR, _ZR)], zbuf)
        pltpu.sync_copy(zbuf, out.at[pl.ds(cid * _N + nbase + k * _ZR, _ZR)])


def _relu(x):
    return jnp.maximum(x, 0.0)


def _dinv_of(degc_ref):
    deg = degc_ref[0:_N] + degc_ref[_N:2 * _N]
    safe = jnp.where(deg > 0, deg, 1.0)
    return jnp.where(deg > 0, 1.0 / jnp.sqrt(safe), 0.0)


def _tc_pre_body(x, w1, b1, w2, b2, w3, b3, gw1, degc, o):
    h = _relu(_mm(x[...], w1[...]) + b1[...])
    h = _relu(_mm(h, w2[...]) + b2[...])
    h = _relu(_mm(h, w3[...]) + b3[...])
    o[...] = _mm(h, gw1[...]) * _dinv_of(degc)


def _tc_mid_body(part, degc, gb, wn, o):
    dinv = _dinv_of(degc)
    raw = part[0:_N] + part[_N:2 * _N]
    h = _relu(raw * dinv + gb[...])
    o[...] = _mm(h, wn[...]) * dinv


def _tc_fin_body(part, degc, gb3, bm, qw1, qb1, qw2, qb2, qw3, qb3, ow, ob, o):
    dinv = _dinv_of(degc)
    raw = part[0:_N] + part[_N:2 * _N]
    h3 = _relu(raw * dinv + gb3[...])
    seg = lax.broadcasted_iota(jnp.int32, (_B, 1), 0)
    pt = (seg == bm[...]).astype(jnp.float32)
    sums = _mm(pt, h3, precision=lax.Precision.HIGHEST)
    cnt = jnp.sum(pt, axis=1, keepdims=True)
    p = sums / jnp.maximum(cnt, 1.0)
    p = _relu(_mm(p, qw1[...]) + qb1[...])
    p = _relu(_mm(p, qw2[...]) + qb2[...])
    p = _relu(_mm(p, qw3[...]) + qb3[...])
    o[...] = _mm(p, ow[...]) + ob[...]


def _tc(body, out_shape, *args):
    return pl.pallas_call(
        body, out_shape=jax.ShapeDtypeStruct(out_shape, jnp.float32)
    )(*args)


def kernel(X, edge_idx, edge_weight, edge_attr, batch_map,
           pW1, pb1, pW2, pb2, pW3, pb3,
           gW1, gb1, gW2, gb2, gW3, gb3,
           qW1, qb1, qW2, qb2, qW3, qb3, oW, ob):
    row = edge_idx[0]
    col = edge_idx[1]
    w = edge_attr.reshape(-1)
    row2 = row.reshape(_E // _C2, _C2)
    col2 = col.reshape(_E // _C2, _C2)
    w2 = w.reshape(_E // _C2, _C2)

    degc = _sc_deg(col2, w2)[:, None]

    hn = _tc(_tc_pre_body, (_N, _H), X,
             pW1, pb1.reshape(1, -1), pW2, pb2.reshape(1, -1),
             pW3, pb3.reshape(1, -1), gW1, degc)
    p1 = _sc_scatter(hn, row2, col2, w2)
    hn = _tc(_tc_mid_body, (_N, _H), p1, degc, gb1.reshape(1, -1), gW2)
    p2 = _sc_scatter(hn, row2, col2, w2)
    hn = _tc(_tc_mid_body, (_N, _H), p2, degc, gb2.reshape(1, -1), gW3)
    p3 = _sc_scatter(hn, row2, col2, w2)

    out = _tc(_tc_fin_body, (_B, 1), p3, degc, gb3.reshape(1, -1),
              batch_map.reshape(1, -1), qW1, qb1.reshape(1, -1),
              qW2, qb2.reshape(1, -1), qW3, qb3.reshape(1, -1),
              oW, ob.reshape(1, -1))
    return out
